# baseline (speedup 1.0000x reference)
import jax
import jax.numpy as jnp
from jax.experimental import pallas as pl

_N = 1024
_NHID = 32
_NHEADS = 4


def _layernorm(x, eps=1e-5):
    m = jnp.mean(x, axis=-1, keepdims=True)
    v = jnp.var(x, axis=-1, keepdims=True)
    return (x - m) / jnp.sqrt(v + eps)


def _elu(x):
    return jnp.where(x > 0, x, jnp.exp(x) - 1.0)


def _gat_layer(x, mask, We_ref, concat):
    outs = []
    ones = jnp.ones((_N, 1), dtype=jnp.float32)
    for i in range(_NHEADS):
        he = jnp.dot(x, We_ref[i], preferred_element_type=jnp.float32)
        h = he[:, :_NHID]
        sd = jnp.exp(he[:, _NHID:_NHID + 4])
        p = sd[:, 0:1]
        r = sd[:, 1:2]
        qt = jnp.transpose(sd[:, 2:4])
        E = jnp.minimum(p * qt[0:1, :], r * qt[1:2, :]) * mask
        h_aug = jnp.concatenate([h, ones], axis=1)
        nd = jnp.dot(E, h_aug, preferred_element_type=jnp.float32)
        hp = nd[:, :_NHID] / nd[:, _NHID:_NHID + 1]
        outs.append(_elu(hp) if concat else hp)
    return jnp.concatenate(outs, axis=1)


def _gat_body(x_in_ref, adj_ref, emb_ref, We1_ref, We2_ref, Wef_ref,
              Wout_ref, bout_ref, out_ref):
    mask = adj_ref[...].astype(jnp.float32)
    x = jnp.dot(x_in_ref[...], emb_ref[...], preferred_element_type=jnp.float32)
    x = _layernorm(x)
    x = _layernorm(_gat_layer(x, mask, We1_ref, True))
    x = _layernorm(_gat_layer(x, mask, We2_ref, True))
    x = _layernorm(_gat_layer(x, mask, Wef_ref, False))
    x = _elu(x)
    logits = jnp.dot(x, Wout_ref[...], preferred_element_type=jnp.float32)
    logits = logits + bout_ref[...][None, :]
    m = jnp.max(logits, axis=1, keepdims=True)
    s = logits - m
    lse = jnp.log(jnp.sum(jnp.exp(s), axis=1, keepdims=True))
    out_ref[...] = s - lse


def _extend_weights(W, a):
    a_src = a[:, 0, :_NHID]
    a_dst = a[:, 0, _NHID:]
    cs = -jnp.einsum('hfk,hk->hf', W, a_src)[..., None]
    cd = -jnp.einsum('hfk,hk->hf', W, a_dst)[..., None]
    return jnp.concatenate([W, cs, 0.2 * cs, cd, 0.2 * cd], axis=2)


def kernel(x_in, adj, emb, W1, a1, W2, a2, Wf, af, Wout, bout):
    We1 = _extend_weights(W1, a1)
    We2 = _extend_weights(W2, a2)
    Wef = _extend_weights(Wf, af)
    return pl.pallas_call(
        _gat_body,
        out_shape=jax.ShapeDtypeStruct((_N, 40), jnp.float32),
    )(x_in, adj, emb, We1, We2, Wef, Wout, bout)


# speedup vs baseline: 1.1406x; 1.1406x over previous
import jax
import jax.numpy as jnp
from jax.experimental import pallas as pl

_N = 1024
_NHID = 32
_NHEADS = 4


def _layernorm(x, eps=1e-5):
    m = jnp.mean(x, axis=-1, keepdims=True)
    v = jnp.var(x, axis=-1, keepdims=True)
    return (x - m) / jnp.sqrt(v + eps)


def _elu(x):
    return jnp.where(x > 0, x, jnp.exp(x) - 1.0)


def _gat_layer(x, mask, W_ref, a_ref, concat):
    outs = []
    ones = jnp.ones((_N, 1), dtype=jnp.float32)
    for i in range(_NHEADS):
        W = W_ref[i]
        a = a_ref[i, 0]
        h = jnp.dot(x, W, preferred_element_type=jnp.float32)
        us = jnp.sum(h * (-a[:_NHID])[None, :], axis=1, keepdims=True)
        ud = jnp.sum(h * (-a[_NHID:])[None, :], axis=1, keepdims=True)
        u = us + jnp.transpose(ud)
        E = jnp.exp(jnp.minimum(u, 0.2 * u)) * mask
        h_aug = jnp.concatenate([h, ones], axis=1)
        nd = jnp.dot(E, h_aug, preferred_element_type=jnp.float32)
        hp = nd[:, :_NHID] / nd[:, _NHID:_NHID + 1]
        outs.append(_elu(hp) if concat else hp)
    return jnp.concatenate(outs, axis=1)


def _gat_body(x_in_ref, adj_ref, emb_ref, W1_ref, a1_ref, W2_ref, a2_ref,
              Wf_ref, af_ref, Wout_ref, bout_ref, out_ref):
    mask = adj_ref[...].astype(jnp.float32)
    x = jnp.dot(x_in_ref[...], emb_ref[...], preferred_element_type=jnp.float32)
    x = _layernorm(x)
    x = _layernorm(_gat_layer(x, mask, W1_ref, a1_ref, True))
    x = _layernorm(_gat_layer(x, mask, W2_ref, a2_ref, True))
    x = _layernorm(_gat_layer(x, mask, Wf_ref, af_ref, False))
    x = _elu(x)
    logits = jnp.dot(x, Wout_ref[...], preferred_element_type=jnp.float32)
    logits = logits + bout_ref[...][None, :]
    m = jnp.max(logits, axis=1, keepdims=True)
    s = logits - m
    lse = jnp.log(jnp.sum(jnp.exp(s), axis=1, keepdims=True))
    out_ref[...] = s - lse


def kernel(x_in, adj, emb, W1, a1, W2, a2, Wf, af, Wout, bout):
    return pl.pallas_call(
        _gat_body,
        out_shape=jax.ShapeDtypeStruct((_N, 40), jnp.float32),
    )(x_in, adj, emb, W1, a1, W2, a2, Wf, af, Wout, bout)
